# Initial kernel scaffold; baseline (speedup 1.0000x reference)
import jax
import jax.numpy as jnp
from jax.experimental import pallas as pl


def kernel(x, mp_adj, edges, index, prev_embs, gc1_W, gc1_b, gc2_W, gc2_b, lin_W, lin_b, weight_lin, bias_lin, w_v, train_s, train_p, ms_logits, ml_W1, ml_b1, ml_W2, ml_b2, ms_W1, ms_b1, ms_W2, ms_b2, red_W, red_b):
    raise NotImplementedError("write your pallas kernel here")


# speedup vs baseline: 5.4141x; 5.4141x over previous
import functools

import jax
import jax.numpy as jnp
from jax import lax
from jax.experimental import pallas as pl
from jax.experimental.pallas import tpu as pltpu
from jax.experimental.pallas import tpu_sc as plsc

N = 10000
E = 320000
B = 100000
H = 128
LG = 64
PROX_W = 0.3

NC, NS = 2, 16
NW = NC * NS
CH = 128

EPAD = NW * 79 * CH
EPW = EPAD // NW
BPAD = NW * 25 * CH
BPW = BPAD // NW
NT = 10240
RPW = NT // NS
TW = 2 * H
BK = 1024

_f32 = jnp.float32


def _wid():
    return lax.axis_index("c") * NS + lax.axis_index("s")


def _sc_deg_sp_body(dst_hbm, idx_hbm, s_hbm, p_hbm, deg_out, s_out, p_out,
                    didx, ones, iidx, svals, pvals, zbuf, acc, sem, sem2):
    cid = lax.axis_index("c")
    sid = lax.axis_index("s")
    wid = _wid()

    @pl.loop(0, CH, step=16)
    def _(i):
        ones[pl.ds(i, 16)] = jnp.ones((16,), _f32)

    @pl.loop(0, RPW, step=16)
    def _(i):
        zbuf[pl.ds(i, 16)] = jnp.zeros((16,), _f32)

    pltpu.sync_copy(zbuf, acc.at[pl.ds(sid * RPW, RPW)])
    plsc.subcore_barrier()

    ebase = wid * EPW

    @pl.loop(0, EPW // CH)
    def _(k):
        off = pl.multiple_of(ebase + k * CH, CH)
        pltpu.sync_copy(dst_hbm.at[pl.ds(off, CH)], didx)
        pltpu.sync_copy(ones, acc.at[didx], add=True)

    bbase = wid * BPW

    @pl.loop(0, BPW // CH)
    def _(k):
        off = pl.multiple_of(bbase + k * CH, CH)
        pltpu.sync_copy(idx_hbm.at[pl.ds(off, CH)], iidx)
        c0 = pltpu.async_copy(s_hbm.at[iidx], svals, sem)
        c1 = pltpu.async_copy(p_hbm.at[iidx], pvals, sem2)
        c0.wait()
        pltpu.sync_copy(svals, s_out.at[pl.ds(off, CH)])
        c1.wait()
        pltpu.sync_copy(pvals, p_out.at[pl.ds(off, CH)])

    plsc.subcore_barrier()
    pltpu.sync_copy(acc.at[pl.ds(sid * RPW, RPW)],
                    deg_out.at[cid, pl.ds(sid * RPW, RPW)])


def _sc_agg_body(g_hbm, src_hbm, dst_hbm, part_out, sidx, didx, rows, zrow,
                 acc, sem):
    cid = lax.axis_index("c")
    sid = lax.axis_index("s")
    wid = _wid()

    @pl.loop(0, CH)
    def _(i):
        for j in range(H // 16):
            zrow[i, pl.ds(j * 16, 16)] = jnp.zeros((16,), _f32)

    for t in range(RPW // CH):
        pltpu.sync_copy(zrow, acc.at[pl.ds(sid * RPW + t * CH, CH)])
    plsc.subcore_barrier()

    ebase = wid * EPW

    @pl.loop(0, EPW // CH)
    def _(k):
        off = pl.multiple_of(ebase + k * CH, CH)
        pltpu.sync_copy(src_hbm.at[pl.ds(off, CH)], sidx)
        pltpu.async_copy(g_hbm.at[sidx], rows, sem).wait()
        pltpu.sync_copy(dst_hbm.at[pl.ds(off, CH)], didx)
        pltpu.sync_copy(rows, acc.at[didx], add=True)

    plsc.subcore_barrier()
    pltpu.sync_copy(acc.at[pl.ds(sid * RPW, RPW)],
                    part_out.at[cid, pl.ds(sid * RPW, RPW)])


def _sc_egather_body(t_hbm, e0_hbm, e1_hbm, a_out, b_out,
                     i0, r0, i1, r1, sem0, sem1):
    wid = _wid()
    bbase = wid * BPW

    @pl.loop(0, BPW // CH)
    def _(k):
        off = pl.multiple_of(bbase + k * CH, CH)
        pltpu.sync_copy(e0_hbm.at[pl.ds(off, CH)], i0)
        pltpu.sync_copy(e1_hbm.at[pl.ds(off, CH)], i1)
        c0 = pltpu.async_copy(t_hbm.at[i0], r0, sem0)
        c1 = pltpu.async_copy(t_hbm.at[i1], r1, sem1)
        c0.wait()
        pltpu.sync_copy(r0, a_out.at[pl.ds(off, CH)])
        c1.wait()
        pltpu.sync_copy(r1, b_out.at[pl.ds(off, CH)])


@functools.cache
def _sc_kernels():
    mesh = plsc.VectorSubcoreMesh(
        core_axis_name="c", subcore_axis_name="s",
        num_cores=NC, num_subcores=NS)
    deg_sp = pl.kernel(
        _sc_deg_sp_body,
        out_type=(
            jax.ShapeDtypeStruct((NC, NT), _f32),
            jax.ShapeDtypeStruct((BPAD,), _f32),
            jax.ShapeDtypeStruct((BPAD,), _f32),
        ),
        mesh=mesh,
        scratch_types=[
            pltpu.VMEM((CH,), jnp.int32),
            pltpu.VMEM((CH,), _f32),
            pltpu.VMEM((CH,), jnp.int32),
            pltpu.VMEM((CH,), _f32),
            pltpu.VMEM((CH,), _f32),
            pltpu.VMEM((RPW,), _f32),
            pltpu.VMEM_SHARED((NT,), _f32),
            pltpu.SemaphoreType.DMA,
            pltpu.SemaphoreType.DMA,
        ],
    )
    agg = pl.kernel(
        _sc_agg_body,
        out_type=jax.ShapeDtypeStruct((NC, NT, H), _f32),
        mesh=mesh,
        scratch_types=[
            pltpu.VMEM((CH,), jnp.int32),
            pltpu.VMEM((CH,), jnp.int32),
            pltpu.VMEM((CH, H), _f32),
            pltpu.VMEM((CH, H), _f32),
            pltpu.VMEM_SHARED((NT, H), _f32),
            pltpu.SemaphoreType.DMA,
        ],
    )
    egather = pl.kernel(
        _sc_egather_body,
        out_type=(
            jax.ShapeDtypeStruct((BPAD, TW), _f32),
            jax.ShapeDtypeStruct((BPAD, TW), _f32),
        ),
        mesh=mesh,
        scratch_types=[
            pltpu.VMEM((CH,), jnp.int32),
            pltpu.VMEM((CH, TW), _f32),
            pltpu.VMEM((CH,), jnp.int32),
            pltpu.VMEM((CH, TW), _f32),
            pltpu.SemaphoreType.DMA,
            pltpu.SemaphoreType.DMA,
        ],
    )
    return deg_sp, agg, egather


_HI = jax.lax.Precision.HIGHEST


def _dot(a, b):
    return jnp.dot(a, b, preferred_element_type=_f32, precision=_HI)


def _k2_body(x_ref, deg_ref, w_ref, dinv_ref, g1_ref):
    d = deg_ref[0, :] + deg_ref[1, :] + 1.0
    dv = lax.rsqrt(d)
    dinv_ref[...] = dv
    g1_ref[...] = _dot(x_ref[...], w_ref[...]) * dv[:, None]


_tc_prep = pl.pallas_call(
    _k2_body,
    grid=(NT // BK,),
    in_specs=[
        pl.BlockSpec((BK, H), lambda i: (i, 0)),
        pl.BlockSpec((NC, BK), lambda i: (0, i)),
        pl.BlockSpec((H, H), lambda i: (0, 0)),
    ],
    out_specs=[
        pl.BlockSpec((BK,), lambda i: (i,)),
        pl.BlockSpec((BK, H), lambda i: (i, 0)),
    ],
    out_shape=[
        jax.ShapeDtypeStruct((NT,), _f32),
        jax.ShapeDtypeStruct((NT, H), _f32),
    ],
)


def _k4_body(part_ref, g1_ref, dinv_ref, b1_ref, w2_ref, g2_ref):
    dv = dinv_ref[...]
    h1 = jnp.tanh((part_ref[0] + part_ref[1] + g1_ref[...]) * dv[:, None]
                  + b1_ref[...])
    g2_ref[...] = _dot(h1, w2_ref[...]) * dv[:, None]


_tc_layer = pl.pallas_call(
    _k4_body,
    grid=(NT // BK,),
    in_specs=[
        pl.BlockSpec((NC, BK, H), lambda i: (0, i, 0)),
        pl.BlockSpec((BK, H), lambda i: (i, 0)),
        pl.BlockSpec((BK,), lambda i: (i,)),
        pl.BlockSpec((H,), lambda i: (0,)),
        pl.BlockSpec((H, H), lambda i: (0, 0)),
    ],
    out_specs=pl.BlockSpec((BK, H), lambda i: (i, 0)),
    out_shape=jax.ShapeDtypeStruct((NT, H), _f32),
)


def _k6_body(part_ref, g2_ref, dinv_ref, b2_ref, wv_ref, linw_ref, linb_ref,
             logits_ref, redw_ref, redb_ref, t_ref):
    dv = dinv_ref[...]
    emb0 = jnp.tanh((part_ref[0] + part_ref[1] + g2_ref[...]) * dv[:, None]
                    + b2_ref[...])
    ae = _dot(emb0, wv_ref[...])
    emb = jnp.tanh(_dot(emb0, linw_ref[:H]) + _dot(ae, linw_ref[H:])
                   + linb_ref[...])
    r = jnp.tanh(_dot(logits_ref[...], redw_ref[...]) + redb_ref[...])
    t_ref[...] = jnp.concatenate([emb, r], axis=1)


_tc_emb = pl.pallas_call(
    _k6_body,
    grid=(NT // BK,),
    in_specs=[
        pl.BlockSpec((NC, BK, H), lambda i: (0, i, 0)),
        pl.BlockSpec((BK, H), lambda i: (i, 0)),
        pl.BlockSpec((BK,), lambda i: (i,)),
        pl.BlockSpec((H,), lambda i: (0,)),
        pl.BlockSpec((H, H), lambda i: (0, 0)),
        pl.BlockSpec((2 * H, H), lambda i: (0, 0)),
        pl.BlockSpec((H,), lambda i: (0,)),
        pl.BlockSpec((BK, LG), lambda i: (i, 0)),
        pl.BlockSpec((LG, H), lambda i: (0, 0)),
        pl.BlockSpec((H,), lambda i: (0,)),
    ],
    out_specs=pl.BlockSpec((BK, TW), lambda i: (i, 0)),
    out_shape=jax.ShapeDtypeStruct((NT, TW), _f32),
)


def _k8_body(a_ref, b_ref, s_ref, p_ref, wl_ref, blin_ref, mlw1_ref, mlb1_ref,
             mlw2_ref, mlb2_ref, msw1_ref, msb1_ref, msw2_ref, msb2_ref,
             out_ref):
    a = a_ref[:, :H]
    ra = a_ref[:, H:]
    b = b_ref[:, :H]
    rb = b_ref[:, H:]
    wl = wl_ref[...]
    sym = (wl + wl.T) * 0.5
    sim = jnp.sum(_dot(a, sym) * b, axis=1) + jnp.sum(blin_ref[...])
    mls = jax.nn.sigmoid(sim)

    mean = (a + b) * 0.5
    mx = jnp.maximum(a, b)
    hml = jnp.tanh(_dot(mean, mlw1_ref[:H]) + _dot(mx, mlw1_ref[H:])
                   + mlb1_ref[...])
    mlw = jnp.tanh(jnp.sum(hml * mlw2_ref[...], axis=1) + mlb2_ref[...])

    rmean = (ra + rb) * 0.5
    rmx = jnp.maximum(ra, rb)
    hms = jnp.tanh(_dot(rmean, msw1_ref[:H]) + _dot(rmx, msw1_ref[H:])
                   + msb1_ref[...])
    msw = jnp.tanh(jnp.sum(hms * msw2_ref[...], axis=1) + msb2_ref[...])

    m = jnp.maximum(jnp.maximum(mlw, msw), PROX_W)
    ea = jnp.exp(mlw - m)
    eb = jnp.exp(msw - m)
    ec = jnp.exp(PROX_W - m)
    z = ea + eb + ec
    s1 = s_ref[...]
    s2 = p_ref[...]
    out_ref[...] = jnp.clip((mls * ea + s1 * eb + s2 * ec) / z, 0.0, 1.0)


_tc_edge = pl.pallas_call(
    _k8_body,
    grid=(BPAD // BK,),
    in_specs=[
        pl.BlockSpec((BK, TW), lambda i: (i, 0)),
        pl.BlockSpec((BK, TW), lambda i: (i, 0)),
        pl.BlockSpec((BK,), lambda i: (i,)),
        pl.BlockSpec((BK,), lambda i: (i,)),
        pl.BlockSpec((H, H), lambda i: (0, 0)),
        pl.BlockSpec((H,), lambda i: (0,)),
        pl.BlockSpec((2 * H, H), lambda i: (0, 0)),
        pl.BlockSpec((H,), lambda i: (0,)),
        pl.BlockSpec((H,), lambda i: (0,)),
        pl.BlockSpec((1,), lambda i: (0,)),
        pl.BlockSpec((2 * H, H), lambda i: (0, 0)),
        pl.BlockSpec((H,), lambda i: (0,)),
        pl.BlockSpec((H,), lambda i: (0,)),
        pl.BlockSpec((1,), lambda i: (0,)),
    ],
    out_specs=pl.BlockSpec((BK,), lambda i: (i,)),
    out_shape=jax.ShapeDtypeStruct((BPAD,), _f32),
)


def kernel(x, mp_adj, edges, index, prev_embs, gc1_W, gc1_b, gc2_W, gc2_b,
           lin_W, lin_b, weight_lin, bias_lin, w_v, train_s, train_p,
           ms_logits, ml_W1, ml_b1, ml_W2, ml_b2, ms_W1, ms_b1, ms_W2, ms_b2,
           red_W, red_b):
    it = mp_adj.dtype
    srcp = jnp.concatenate([mp_adj[0], jnp.zeros((EPAD - E,), it)])
    dstp = jnp.concatenate([mp_adj[1], jnp.full((EPAD - E,), N, it)])
    e0p = jnp.concatenate([edges[0], jnp.zeros((BPAD - B,), edges.dtype)])
    e1p = jnp.concatenate([edges[1], jnp.zeros((BPAD - B,), edges.dtype)])
    idxp = jnp.concatenate([index, jnp.zeros((BPAD - B,), index.dtype)])
    x_pad = jnp.pad(x, ((0, NT - N), (0, 0)))
    logits_pad = jnp.pad(ms_logits, ((0, NT - N), (0, 0)))

    sc_deg_sp, sc_agg, sc_egather = _sc_kernels()
    deg_parts, s_g, p_g = sc_deg_sp(dstp, idxp, train_s, train_p)
    dinv, g1 = _tc_prep(x_pad, deg_parts, gc1_W)
    part1 = sc_agg(g1, srcp, dstp)
    g2 = _tc_layer(part1, g1, dinv, gc1_b, gc2_W)
    part2 = sc_agg(g2, srcp, dstp)
    t_tab = _tc_emb(part2, g2, dinv, gc2_b, w_v, lin_W, lin_b, logits_pad,
                    red_W, red_b)
    a_rows, b_rows = sc_egather(t_tab, e0p, e1p)
    final = _tc_edge(a_rows, b_rows, s_g, p_g, weight_lin, bias_lin,
                     ml_W1, ml_b1, ml_W2[:, 0], ml_b2, ms_W1, ms_b1,
                     ms_W2[:, 0], ms_b2)
    return final[:B]
